# baseline (speedup 1.0000x reference)
import functools

import jax
import jax.numpy as jnp
import numpy as np
from jax import lax
from jax.experimental import pallas as pl
from jax.experimental.pallas import tpu as pltpu
from jax.experimental.pallas import tpu_sc as plsc

_RATE = 0.1

_drop_cache = {}


def _drop_vec(b):
    if b not in _drop_cache:
        keep = 1.0 - _RATE
        with jax.ensure_compile_time_eval():
            u = jax.random.uniform(jax.random.key(1), (b, 1), dtype=jnp.float32)
            drop = jnp.ones((b, 1), jnp.float32) / keep * jnp.floor(keep + u)
        _drop_cache[b] = np.asarray(drop)[:, 0]
    return _drop_cache[b]


def _sc_scale_body(nn_hbm, scale_hbm, nn_v, sc_v, *, dd, per_w):
    pltpu.sync_copy(nn_hbm, nn_v)
    wid = lax.axis_index("s") * 2 + lax.axis_index("c")
    base = wid * per_w
    nn = nn_v[...]
    steps = []
    acc = nn[0]
    for k in range(1, len(dd)):
        if dd[k] != 0.0:
            steps.append((acc, dd[k]))
        if k < len(dd) - 1:
            acc = acc + nn[k]
    for j in range(per_w // 16):
        r = lax.iota(jnp.int32, 16) + (base + 16 * j)
        s = jnp.full((16,), dd[0], jnp.float32)
        for e_k, dd_k in steps:
            s = s + jnp.where(r >= e_k, jnp.float32(dd_k), jnp.float32(0.0))
        sc_v[pl.ds(16 * j, 16)] = s
    pltpu.sync_copy(sc_v, scale_hbm.at[pl.ds(base, per_w)])


def _mul_body(sc_ref, x_ref, o_ref):
    o_ref[...] = x_ref[...] * sc_ref[...][:, :, None]


def kernel(inputs, n_node):
    n, d = inputs.shape
    b = n_node.shape[0]
    drop = _drop_vec(b)
    dd = [float(drop[0])] + [float(drop[k] - drop[k - 1]) for k in range(1, b)]

    nw = 32
    per_w = n // nw
    mesh = plsc.VectorSubcoreMesh(core_axis_name="c", subcore_axis_name="s")
    scale = pl.kernel(
        functools.partial(_sc_scale_body, dd=tuple(dd), per_w=per_w),
        out_type=jax.ShapeDtypeStruct((n,), jnp.float32),
        mesh=mesh,
        scratch_types=[
            pltpu.VMEM((b,), jnp.int32),
            pltpu.VMEM((per_w,), jnp.float32),
        ],
    )(n_node.astype(jnp.int32))

    rows_per_blk = 8192
    grid = n // rows_per_blk
    x3 = inputs.reshape(n // 128, 128, d)
    out = pl.pallas_call(
        _mul_body,
        grid=(grid,),
        in_specs=[
            pl.BlockSpec((rows_per_blk // 128, 128), lambda i: (i, 0)),
            pl.BlockSpec((rows_per_blk // 128, 128, d), lambda i: (i, 0, 0)),
        ],
        out_specs=pl.BlockSpec((rows_per_blk // 128, 128, d), lambda i: (i, 0, 0)),
        out_shape=jax.ShapeDtypeStruct((n // 128, 128, d), inputs.dtype),
    )(scale.reshape(n // 128, 128), x3)
    return out.reshape(n, d)
